# baseline (speedup 1.0000x reference)
import functools

import jax
import jax.numpy as jnp
from jax import lax
from jax.experimental import pallas as pl
from jax.experimental.pallas import tpu as pltpu
from jax.experimental.pallas import tpu_sc as plsc

_NC = 2
_NS = 16
_L = 16
_NW = _NC * _NS

_N, _P, _D = 8, 4096, 3
_NCH = _NW // _N
_CHUNK = _P // _NCH
_IP = 4
_BIG = 3.0e38


def _round_to_bf16(v):
    s = v * jnp.float32(65537.0)
    return s - (s - v)


def _chamfer_partials_sc(xt, yt):
    mesh = plsc.VectorSubcoreMesh(
        core_axis_name="c", subcore_axis_name="s",
        num_cores=_NC, num_subcores=_NS)

    @functools.partial(
        pl.kernel,
        out_type=(
            jax.ShapeDtypeStruct((_NW, _CHUNK * _L), jnp.float32),
            jax.ShapeDtypeStruct((_N, _NCH, _P), jnp.float32),
        ),
        mesh=mesh,
        scratch_types=[
            pltpu.VMEM((_D * _CHUNK,), jnp.float32),
            pltpu.VMEM((_D * _P,), jnp.float32),
            pltpu.VMEM((_P,), jnp.float32),
            pltpu.VMEM((_P,), jnp.float32),
            pltpu.VMEM((_CHUNK * _L,), jnp.float32),
        ],
    )
    def body(xt_hbm, yt_hbm, rowmin_hbm, colmin_hbm, x_v, y_v, yy_v, col_v,
             rm_v):
        wid = lax.axis_index("c") * _NS + lax.axis_index("s")
        b = wid // _NCH
        ch = wid % _NCH

        for d in range(_D):
            pltpu.sync_copy(
                xt_hbm.at[b, pl.ds(d * _P + ch * _CHUNK, _CHUNK)],
                x_v.at[pl.ds(d * _CHUNK, _CHUNK)])
        pltpu.sync_copy(yt_hbm.at[b], y_v)

        def prep(q, carry):
            y0 = y_v[pl.ds(q * _L, _L)]
            y1 = y_v[pl.ds(_P + q * _L, _L)]
            y2 = y_v[pl.ds(2 * _P + q * _L, _L)]
            yy_v[pl.ds(q * _L, _L)] = y0 * y0 + y1 * y1 + y2 * y2
            y_v[pl.ds(q * _L, _L)] = _round_to_bf16(y0)
            y_v[pl.ds(_P + q * _L, _L)] = _round_to_bf16(y1)
            y_v[pl.ds(2 * _P + q * _L, _L)] = _round_to_bf16(y2)
            col_v[pl.ds(q * _L, _L)] = jnp.full((_L,), _BIG, jnp.float32)
            return carry

        lax.fori_loop(0, _P // _L, prep, 0)

        def g_body(g, carry):
            vx0 = x_v[pl.ds(g * _L, _L)]
            vx1 = x_v[pl.ds(_CHUNK + g * _L, _L)]
            vx2 = x_v[pl.ds(2 * _CHUNK + g * _L, _L)]
            vm0 = -2.0 * _round_to_bf16(vx0)
            vm1 = -2.0 * _round_to_bf16(vx1)
            vm2 = -2.0 * _round_to_bf16(vx2)
            vxx = vx0 * vx0 + vx1 * vx1 + vx2 * vx2
            for t in range(_L // _IP):
                m0 = [jnp.full((_L,), vm0[t * _IP + i]) for i in range(_IP)]
                m1 = [jnp.full((_L,), vm1[t * _IP + i]) for i in range(_IP)]
                m2 = [jnp.full((_L,), vm2[t * _IP + i]) for i in range(_IP)]
                xx = [jnp.full((_L,), vxx[t * _IP + i]) for i in range(_IP)]

                def q_body(q, rms, m0=m0, m1=m1, m2=m2, xx=xx):
                    y0 = y_v[pl.ds(q * _L, _L)]
                    y1 = y_v[pl.ds(_P + q * _L, _L)]
                    y2 = y_v[pl.ds(2 * _P + q * _L, _L)]
                    yy = yy_v[pl.ds(q * _L, _L)]
                    c = col_v[pl.ds(q * _L, _L)]
                    out = []
                    for i in range(_IP):
                        d = (yy + xx[i]) + (m0[i] * y0 + m1[i] * y1
                                            + m2[i] * y2)
                        out.append(jnp.minimum(rms[i], d))
                        c = jnp.minimum(c, d)
                    col_v[pl.ds(q * _L, _L)] = c
                    return tuple(out)

                rms = lax.fori_loop(
                    0, _P // _L, q_body,
                    tuple(jnp.full((_L,), _BIG, jnp.float32)
                          for _ in range(_IP)))
                for i in range(_IP):
                    rm_v[pl.ds((g * _L + t * _IP + i) * _L, _L)] = rms[i]
            return carry

        lax.fori_loop(0, _CHUNK // _L, g_body, 0)

        pltpu.sync_copy(rm_v, rowmin_hbm.at[wid])
        pltpu.sync_copy(col_v, colmin_hbm.at[b, ch])

    return body(xt, yt)


def _combine_tc(rowmin, colmin):
    def body(rm_ref, cm_ref, out_ref):
        rm = jnp.min(rm_ref[...], axis=-1)
        cm = jnp.min(cm_ref[...], axis=1)
        out_ref[0, 0] = (jnp.sum(rm) + jnp.sum(cm)) / jnp.float32(_N * _P)

    out = pl.pallas_call(
        body,
        out_shape=jax.ShapeDtypeStruct((1, 1), jnp.float32),
        out_specs=pl.BlockSpec(memory_space=pltpu.SMEM),
    )(rowmin, colmin)
    return out.reshape(())


def kernel(x, y):
    xt = jnp.transpose(x, (0, 2, 1)).reshape(_N, _D * _P)
    yt = jnp.transpose(y, (0, 2, 1)).reshape(_N, _D * _P)
    rowmin, colmin = _chamfer_partials_sc(xt, yt)
    rowmin = rowmin.reshape(_NW * _CHUNK, _L)
    return _combine_tc(rowmin, colmin)


# speedup vs baseline: 3.5501x; 3.5501x over previous
import functools

import jax
import jax.numpy as jnp
from jax import lax
from jax.experimental import pallas as pl
from jax.experimental.pallas import tpu as pltpu
from jax.experimental.pallas import tpu_sc as plsc

_NC = 2
_NS = 16
_L = 16
_NW = _NC * _NS

_N, _P, _D = 8, 4096, 3
_NB_SC = 2
_NB_TC = _N - _NB_SC
_NCH = _NW // _NB_SC
_CHUNK = _P // _NCH
_IP = 4
_BP = 512
_BIG = 3.0e38


def _round_to_bf16(v):
    s = v * jnp.float32(65537.0)
    return s - (s - v)


def _chamfer_partials_sc(xt, yt):
    mesh = plsc.VectorSubcoreMesh(
        core_axis_name="c", subcore_axis_name="s",
        num_cores=_NC, num_subcores=_NS)

    @functools.partial(
        pl.kernel,
        out_type=(
            jax.ShapeDtypeStruct((_NW, _CHUNK * _L), jnp.float32),
            jax.ShapeDtypeStruct((_NB_SC, _NCH, _P), jnp.float32),
        ),
        mesh=mesh,
        scratch_types=[
            pltpu.VMEM((_D * _CHUNK,), jnp.float32),
            pltpu.VMEM((_D * _P,), jnp.float32),
            pltpu.VMEM((_P,), jnp.float32),
            pltpu.VMEM((_P,), jnp.float32),
            pltpu.VMEM((_CHUNK * _L,), jnp.float32),
        ],
    )
    def body(xt_hbm, yt_hbm, rowmin_hbm, colmin_hbm, x_v, y_v, yy_v, col_v,
             rm_v):
        wid = lax.axis_index("c") * _NS + lax.axis_index("s")
        b = wid // _NCH
        ch = wid % _NCH

        for d in range(_D):
            pltpu.sync_copy(
                xt_hbm.at[b, pl.ds(d * _P + ch * _CHUNK, _CHUNK)],
                x_v.at[pl.ds(d * _CHUNK, _CHUNK)])
        pltpu.sync_copy(yt_hbm.at[b], y_v)

        def prep(q, carry):
            y0 = y_v[pl.ds(q * _L, _L)]
            y1 = y_v[pl.ds(_P + q * _L, _L)]
            y2 = y_v[pl.ds(2 * _P + q * _L, _L)]
            yy_v[pl.ds(q * _L, _L)] = y0 * y0 + y1 * y1 + y2 * y2
            y_v[pl.ds(q * _L, _L)] = _round_to_bf16(y0)
            y_v[pl.ds(_P + q * _L, _L)] = _round_to_bf16(y1)
            y_v[pl.ds(2 * _P + q * _L, _L)] = _round_to_bf16(y2)
            col_v[pl.ds(q * _L, _L)] = jnp.full((_L,), _BIG, jnp.float32)
            return carry

        lax.fori_loop(0, _P // _L, prep, 0)

        def g_body(g, carry):
            vx0 = x_v[pl.ds(g * _L, _L)]
            vx1 = x_v[pl.ds(_CHUNK + g * _L, _L)]
            vx2 = x_v[pl.ds(2 * _CHUNK + g * _L, _L)]
            vm0 = -2.0 * _round_to_bf16(vx0)
            vm1 = -2.0 * _round_to_bf16(vx1)
            vm2 = -2.0 * _round_to_bf16(vx2)
            vxx = vx0 * vx0 + vx1 * vx1 + vx2 * vx2
            for t in range(_L // _IP):
                m0 = [jnp.full((_L,), vm0[t * _IP + i]) for i in range(_IP)]
                m1 = [jnp.full((_L,), vm1[t * _IP + i]) for i in range(_IP)]
                m2 = [jnp.full((_L,), vm2[t * _IP + i]) for i in range(_IP)]
                xx = [jnp.full((_L,), vxx[t * _IP + i]) for i in range(_IP)]

                def q_body(q, rms, m0=m0, m1=m1, m2=m2, xx=xx):
                    y0 = y_v[pl.ds(q * _L, _L)]
                    y1 = y_v[pl.ds(_P + q * _L, _L)]
                    y2 = y_v[pl.ds(2 * _P + q * _L, _L)]
                    yy = yy_v[pl.ds(q * _L, _L)]
                    c = col_v[pl.ds(q * _L, _L)]
                    out = []
                    for i in range(_IP):
                        d = (yy + xx[i]) + (m0[i] * y0 + m1[i] * y1
                                            + m2[i] * y2)
                        out.append(jnp.minimum(rms[i], d))
                        c = jnp.minimum(c, d)
                    col_v[pl.ds(q * _L, _L)] = c
                    return tuple(out)

                rms = lax.fori_loop(
                    0, _P // _L, q_body,
                    tuple(jnp.full((_L,), _BIG, jnp.float32)
                          for _ in range(_IP)))
                for i in range(_IP):
                    rm_v[pl.ds((g * _L + t * _IP + i) * _L, _L)] = rms[i]
            return carry

        lax.fori_loop(0, _CHUNK // _L, g_body, 0)

        pltpu.sync_copy(rm_v, rowmin_hbm.at[wid])
        pltpu.sync_copy(col_v, colmin_hbm.at[b, ch])

    return body(xt, yt)


def _chamfer_partials_tc(x, yt3):
    def body(x_ref, yt_ref, rs_ref, cm_ref):
        j = pl.program_id(1)
        xblk = x_ref[0]
        yt = yt_ref[0]
        xx = jnp.sum(xblk * xblk, axis=1)
        yy = jnp.sum(yt * yt, axis=0)
        ab = lax.dot_general(
            xblk.astype(jnp.bfloat16), yt.astype(jnp.bfloat16),
            (((1,), (0,)), ((), ())),
            preferred_element_type=jnp.float32)
        d = (xx[:, None] + yy[None, :]) - 2.0 * ab

        @pl.when(j == 0)
        def _init():
            rs_ref[...] = jnp.zeros((1, 8, 128), jnp.float32)
            cm_ref[...] = jnp.full((1, 8, _P), _BIG, jnp.float32)

        rs_ref[0, 0, :] += jnp.full((128,), jnp.sum(jnp.min(d, axis=1)))
        cm_ref[0, 0, :] = jnp.minimum(cm_ref[0, 0, :], jnp.min(d, axis=0))

    return pl.pallas_call(
        body,
        grid=(_NB_TC, _P // _BP),
        in_specs=[
            pl.BlockSpec((1, _BP, _D), lambda b, j: (b + _NB_SC, j, 0)),
            pl.BlockSpec((1, _D, _P), lambda b, j: (b + _NB_SC, 0, 0)),
        ],
        out_specs=[
            pl.BlockSpec((1, 8, 128), lambda b, j: (b, 0, 0)),
            pl.BlockSpec((1, 8, _P), lambda b, j: (b, 0, 0)),
        ],
        out_shape=[
            jax.ShapeDtypeStruct((_NB_TC, 8, 128), jnp.float32),
            jax.ShapeDtypeStruct((_NB_TC, 8, _P), jnp.float32),
        ],
        compiler_params=pltpu.CompilerParams(
            dimension_semantics=("arbitrary", "arbitrary")),
    )(x, yt3)


def _combine_tc(sc_rowmin, sc_colmin, tc_rowsum, tc_colmin):
    def body(srm_ref, scm_ref, trs_ref, tcm_ref, out_ref):
        srm = jnp.min(srm_ref[...], axis=-1)
        scm = jnp.min(scm_ref[...], axis=1)
        total = (jnp.sum(srm) + jnp.sum(scm)
                 + jnp.sum(trs_ref[:, 0, 0]) + jnp.sum(tcm_ref[:, 0, :]))
        out_ref[0, 0] = total / jnp.float32(_N * _P)

    out = pl.pallas_call(
        body,
        out_shape=jax.ShapeDtypeStruct((1, 1), jnp.float32),
        out_specs=pl.BlockSpec(memory_space=pltpu.SMEM),
    )(sc_rowmin, sc_colmin, tc_rowsum, tc_colmin)
    return out.reshape(())


def kernel(x, y):
    xt3 = jnp.transpose(x, (0, 2, 1))
    yt3 = jnp.transpose(y, (0, 2, 1))
    sc_rowmin, sc_colmin = _chamfer_partials_sc(
        xt3.reshape(_N, _D * _P), yt3.reshape(_N, _D * _P))
    tc_rowsum, tc_colmin = _chamfer_partials_tc(x, yt3)
    sc_rowmin = sc_rowmin.reshape(_NW * _CHUNK, _L)
    return _combine_tc(sc_rowmin, sc_colmin, tc_rowsum, tc_colmin)


# speedup vs baseline: 5.4986x; 1.5489x over previous
import functools

import jax
import jax.numpy as jnp
from jax import lax
from jax.experimental import pallas as pl
from jax.experimental.pallas import tpu as pltpu
from jax.experimental.pallas import tpu_sc as plsc

_NC = 2
_NS = 16
_L = 16
_NW = _NC * _NS

_N, _P, _D = 8, 4096, 3
_RSC = 512
_NCH = _NW // _N
_CHUNK = _RSC // _NCH
_IP = 4
_UQ = 2
_BP = 512
_JTC = (_P - _RSC) // _BP
_BIG = 3.0e38


def _round_to_bf16(v):
    s = v * jnp.float32(65537.0)
    return s - (s - v)


def _chamfer_partials_sc(xt, yt):
    mesh = plsc.VectorSubcoreMesh(
        core_axis_name="c", subcore_axis_name="s",
        num_cores=_NC, num_subcores=_NS)

    @functools.partial(
        pl.kernel,
        out_type=(
            jax.ShapeDtypeStruct((_NW, _CHUNK * _L), jnp.float32),
            jax.ShapeDtypeStruct((_N, _NCH, _P), jnp.float32),
        ),
        mesh=mesh,
        scratch_types=[
            pltpu.VMEM((_D * _CHUNK,), jnp.float32),
            pltpu.VMEM((_D * _P,), jnp.float32),
            pltpu.VMEM((_P,), jnp.float32),
            pltpu.VMEM((_P,), jnp.float32),
            pltpu.VMEM((_CHUNK * _L,), jnp.float32),
        ],
    )
    def body(xt_hbm, yt_hbm, rowmin_hbm, colmin_hbm, x_v, y_v, yy_v, col_v,
             rm_v):
        wid = lax.axis_index("c") * _NS + lax.axis_index("s")
        b = wid // _NCH
        ch = wid % _NCH

        for d in range(_D):
            pltpu.sync_copy(
                xt_hbm.at[b, pl.ds(d * _P + ch * _CHUNK, _CHUNK)],
                x_v.at[pl.ds(d * _CHUNK, _CHUNK)])
        pltpu.sync_copy(yt_hbm.at[b], y_v)

        def prep(q, carry):
            y0 = y_v[pl.ds(q * _L, _L)]
            y1 = y_v[pl.ds(_P + q * _L, _L)]
            y2 = y_v[pl.ds(2 * _P + q * _L, _L)]
            yy_v[pl.ds(q * _L, _L)] = y0 * y0 + y1 * y1 + y2 * y2
            y_v[pl.ds(q * _L, _L)] = _round_to_bf16(y0)
            y_v[pl.ds(_P + q * _L, _L)] = _round_to_bf16(y1)
            y_v[pl.ds(2 * _P + q * _L, _L)] = _round_to_bf16(y2)
            col_v[pl.ds(q * _L, _L)] = jnp.full((_L,), _BIG, jnp.float32)
            return carry

        lax.fori_loop(0, _P // _L, prep, 0)

        def g_body(g, carry):
            vx0 = x_v[pl.ds(g * _L, _L)]
            vx1 = x_v[pl.ds(_CHUNK + g * _L, _L)]
            vx2 = x_v[pl.ds(2 * _CHUNK + g * _L, _L)]
            vm0 = -2.0 * _round_to_bf16(vx0)
            vm1 = -2.0 * _round_to_bf16(vx1)
            vm2 = -2.0 * _round_to_bf16(vx2)
            vxx = vx0 * vx0 + vx1 * vx1 + vx2 * vx2
            for t in range(_L // _IP):
                m0 = [jnp.full((_L,), vm0[t * _IP + i]) for i in range(_IP)]
                m1 = [jnp.full((_L,), vm1[t * _IP + i]) for i in range(_IP)]
                m2 = [jnp.full((_L,), vm2[t * _IP + i]) for i in range(_IP)]
                xx = [jnp.full((_L,), vxx[t * _IP + i]) for i in range(_IP)]

                def q_body(q, rms, m0=m0, m1=m1, m2=m2, xx=xx):
                    out = list(rms)
                    for u in range(_UQ):
                        base = (q * _UQ + u) * _L
                        y0 = y_v[pl.ds(base, _L)]
                        y1 = y_v[pl.ds(_P + base, _L)]
                        y2 = y_v[pl.ds(2 * _P + base, _L)]
                        yy = yy_v[pl.ds(base, _L)]
                        ds = []
                        for i in range(_IP):
                            d = (yy + xx[i]) + (m0[i] * y0 + m1[i] * y1
                                                + m2[i] * y2)
                            out[i] = jnp.minimum(out[i], d)
                            ds.append(d)
                        while len(ds) > 1:
                            ds = [jnp.minimum(ds[k], ds[k + 1])
                                  for k in range(0, len(ds) - 1, 2)] + (
                                      [ds[-1]] if len(ds) % 2 else [])
                        col_v[pl.ds(base, _L)] = jnp.minimum(
                            col_v[pl.ds(base, _L)], ds[0])
                    return tuple(out)

                rms = lax.fori_loop(
                    0, _P // (_L * _UQ), q_body,
                    tuple(jnp.full((_L,), _BIG, jnp.float32)
                          for _ in range(_IP)))
                for i in range(_IP):
                    rm_v[pl.ds((g * _L + t * _IP + i) * _L, _L)] = rms[i]
            return carry

        lax.fori_loop(0, _CHUNK // _L, g_body, 0)

        pltpu.sync_copy(rm_v, rowmin_hbm.at[wid])
        pltpu.sync_copy(col_v, colmin_hbm.at[b, ch])

    return body(xt, yt)


def _chamfer_partials_tc(x, yt3):
    def body(x_ref, yt_ref, rs_ref, cm_ref):
        j = pl.program_id(1)
        xblk = x_ref[0]
        yt = yt_ref[0]
        xx = jnp.sum(xblk * xblk, axis=1)
        yy = jnp.sum(yt * yt, axis=0)
        ab2 = lax.dot_general(
            (xblk * -2.0).astype(jnp.bfloat16), yt.astype(jnp.bfloat16),
            (((1,), (0,)), ((), ())),
            preferred_element_type=jnp.float32)
        d = (xx[:, None] + yy[None, :]) + ab2

        @pl.when(j == 0)
        def _init():
            rs_ref[...] = jnp.zeros((1, 8, 128), jnp.float32)
            cm_ref[...] = jnp.full((1, 8, _P), _BIG, jnp.float32)

        rs_ref[0, 0, :] += jnp.full((128,), jnp.sum(jnp.min(d, axis=1)))
        cm_ref[0, 0, :] = jnp.minimum(cm_ref[0, 0, :], jnp.min(d, axis=0))

    return pl.pallas_call(
        body,
        grid=(_N, _JTC),
        in_specs=[
            pl.BlockSpec((1, _BP, _D), lambda b, j: (b, _RSC // _BP + j, 0)),
            pl.BlockSpec((1, _D, _P), lambda b, j: (b, 0, 0)),
        ],
        out_specs=[
            pl.BlockSpec((1, 8, 128), lambda b, j: (b, 0, 0)),
            pl.BlockSpec((1, 8, _P), lambda b, j: (b, 0, 0)),
        ],
        out_shape=[
            jax.ShapeDtypeStruct((_N, 8, 128), jnp.float32),
            jax.ShapeDtypeStruct((_N, 8, _P), jnp.float32),
        ],
        compiler_params=pltpu.CompilerParams(
            dimension_semantics=("arbitrary", "arbitrary")),
    )(x, yt3)


def _combine_tc(sc_rowmin, sc_colmin, tc_rowsum, tc_colmin):
    def body(srm_ref, scm_ref, trs_ref, tcm_ref, out_ref):
        srm = jnp.min(srm_ref[...], axis=-1)
        scm = jnp.min(scm_ref[...], axis=1)
        cm = jnp.minimum(scm, tcm_ref[:, 0, :])
        total = jnp.sum(srm) + jnp.sum(cm) + jnp.sum(trs_ref[:, 0, 0])
        out_ref[0, 0] = total / jnp.float32(_N * _P)

    out = pl.pallas_call(
        body,
        out_shape=jax.ShapeDtypeStruct((1, 1), jnp.float32),
        out_specs=pl.BlockSpec(memory_space=pltpu.SMEM),
    )(sc_rowmin, sc_colmin, tc_rowsum, tc_colmin)
    return out.reshape(())


def kernel(x, y):
    xt3 = jnp.transpose(x, (0, 2, 1))
    yt3 = jnp.transpose(y, (0, 2, 1))
    sc_rowmin, sc_colmin = _chamfer_partials_sc(
        xt3.reshape(_N, _D * _P), yt3.reshape(_N, _D * _P))
    tc_rowsum, tc_colmin = _chamfer_partials_tc(x, yt3)
    sc_rowmin = sc_rowmin.reshape(_NW * _CHUNK, _L)
    return _combine_tc(sc_rowmin, sc_colmin, tc_rowsum, tc_colmin)
